# baseline (speedup 1.0000x reference)
import jax
import jax.numpy as jnp
from jax import lax
from jax.experimental import pallas as pl
from jax.experimental.pallas import tpu as pltpu

_B, _P, _C, _K = 32, 8732, 21, 12
_THRESHOLD = 0.5
_NEG_POS_RATIO = 3
_ALPHA = 1.0


def _match_ce_body(priors_ref, boxes_ref, labels_ref,
                   locs1_ref, scores1_ref, locs2_ref, scores2_ref,
                   cn1_ref, cn2_ref, part_ref):
    f32 = jnp.float32
    K, P, C = _K, _P, _C

    pcx = priors_ref[0:1, :]
    pcy = priors_ref[1:2, :]
    pw = priors_ref[2:3, :]
    ph = priors_ref[3:4, :]
    px0 = pcx - pw * 0.5
    py0 = pcy - ph * 0.5
    px1 = pcx + pw * 0.5
    py1 = pcy + ph * 0.5

    b = boxes_ref[0]
    labs = labels_ref[0]
    bx0 = b[:, 0:1]
    by0 = b[:, 1:2]
    bx1 = b[:, 2:3]
    by1 = b[:, 3:4]

    iw = jnp.maximum(jnp.minimum(bx1, px1) - jnp.maximum(bx0, px0), 0.0)
    ih = jnp.maximum(jnp.minimum(by1, py1) - jnp.maximum(by0, py0), 0.0)
    inter = iw * ih
    area_b = (bx1 - bx0) * (by1 - by0)
    area_p = (px1 - px0) * (py1 - py0)
    ov = inter / (area_b + area_p - inter)

    kiota = lax.broadcasted_iota(jnp.int32, (K, P), 0)
    piota = lax.broadcasted_iota(jnp.int32, (K, P), 1)

    ofep = jnp.max(ov, axis=0, keepdims=True)
    obj = jnp.min(jnp.where(ov == ofep, kiota, K), axis=0, keepdims=True)
    mk = jnp.max(ov, axis=1, keepdims=True)
    pfeo = jnp.min(jnp.where(ov == mk, piota, P), axis=1, keepdims=True)
    win = jnp.max(jnp.where(pfeo == piota, kiota, -1), axis=0, keepdims=True)
    obj = jnp.where(win >= 0, win, obj)
    ovf = jnp.where(win >= 0, jnp.full_like(ofep, 1.0), ofep)

    onehot = obj == kiota
    lab_sel = jnp.sum(jnp.where(onehot, labs, 0), axis=0, keepdims=True)
    label = jnp.where(ovf < _THRESHOLD, 0, lab_sel)
    pos = label > 0
    posf = pos.astype(f32)
    npos = jnp.sum(posf)

    onehotf = onehot.astype(f32)
    gx0 = jnp.sum(onehotf * bx0, axis=0, keepdims=True)
    gy0 = jnp.sum(onehotf * by0, axis=0, keepdims=True)
    gx1 = jnp.sum(onehotf * bx1, axis=0, keepdims=True)
    gy1 = jnp.sum(onehotf * by1, axis=0, keepdims=True)
    cx = (gx0 + gx1) * 0.5
    cy = (gy0 + gy1) * 0.5
    w = gx1 - gx0
    h = gy1 - gy0
    tl0 = (cx - pcx) / (pw / 10.0)
    tl1 = (cy - pcy) / (ph / 10.0)
    tl2 = jnp.log(w / pw) * 5.0
    tl3 = jnp.log(h / ph) * 5.0

    ciota = lax.broadcasted_iota(jnp.int32, (C, P), 0)

    def branch(locs_ref, scores_ref, cn_ref):
        loc_abs = (jnp.abs(locs_ref[0, 0:1, :] - tl0)
                   + jnp.abs(locs_ref[0, 1:2, :] - tl1)
                   + jnp.abs(locs_ref[0, 2:3, :] - tl2)
                   + jnp.abs(locs_ref[0, 3:4, :] - tl3))
        loc_sum = jnp.sum(loc_abs * posf)
        s = scores_ref[0].T
        mx = jnp.max(s, axis=0, keepdims=True)
        lse = jnp.log(jnp.sum(jnp.exp(s - mx), axis=0, keepdims=True)) + mx
        strue = jnp.sum(jnp.where(ciota == label, s, 0.0), axis=0, keepdims=True)
        ce = lse - strue
        cep = jnp.sum(ce * posf)
        cn = jnp.maximum(jnp.where(pos, 0.0, ce), 0.0)
        cn_ref[0] = cn
        return loc_sum, cep

    l1, c1 = branch(locs1_ref, scores1_ref, cn1_ref)
    l2, c2 = branch(locs2_ref, scores2_ref, cn2_ref)

    lane = lax.broadcasted_iota(jnp.int32, (1, 128), 1)
    row = (npos * (lane == 0).astype(f32)
           + l1 * (lane == 1).astype(f32)
           + l2 * (lane == 2).astype(f32)
           + c1 * (lane == 3).astype(f32)
           + c2 * (lane == 4).astype(f32))
    part_ref[0] = row


def _hardneg_body(cn1_ref, cn2_ref, part_ref, out1_ref, out2_ref):
    f32 = jnp.float32
    parts = part_ref[...]
    npos = parts[:, 0:1]
    l1_tot = jnp.sum(parts[:, 1:2])
    l2_tot = jnp.sum(parts[:, 2:3])
    c1_tot = jnp.sum(parts[:, 3:4])
    c2_tot = jnp.sum(parts[:, 4:5])
    np_tot = jnp.sum(npos)
    m = npos * float(_NEG_POS_RATIO)

    def topm_sum(v):
        t = jnp.zeros((_B, 1), jnp.int32)
        for bit in range(30, -1, -1):
            cand = t | (1 << bit)
            tf = lax.bitcast_convert_type(cand, f32)
            cnt = jnp.sum((v >= tf).astype(f32), axis=1, keepdims=True)
            t = jnp.where(cnt >= m, cand, t)
        tf = lax.bitcast_convert_type(t, f32)
        gtf = (v > tf).astype(f32)
        cnt_gt = jnp.sum(gtf, axis=1, keepdims=True)
        hard = jnp.sum(v * gtf, axis=1, keepdims=True) + (m - cnt_gt) * tf
        return jnp.sum(hard)

    h1 = topm_sum(cn1_ref[...])
    h2 = topm_sum(cn2_ref[...])
    o1 = (h1 + c1_tot) / np_tot + _ALPHA * l1_tot / (np_tot * 4.0)
    o2 = (h2 + c2_tot) / np_tot + _ALPHA * l2_tot / (np_tot * 4.0)
    out1_ref[...] = o1.reshape(1, 1)
    out2_ref[...] = o2.reshape(1, 1)


def kernel(predicted_locs1, predicted_scores1, predicted_locs2,
           predicted_scores2, boxes, labels, priors_cxcy):
    B, P, C, K = _B, _P, _C, _K
    priors_t = priors_cxcy.T
    locs1_t = jnp.transpose(predicted_locs1, (0, 2, 1))
    locs2_t = jnp.transpose(predicted_locs2, (0, 2, 1))
    scores1_t = predicted_scores1
    scores2_t = predicted_scores2
    labels3 = labels.astype(jnp.int32).reshape(B, K, 1)

    cn1, cn2, part = pl.pallas_call(
        _match_ce_body,
        grid=(B,),
        in_specs=[
            pl.BlockSpec((4, P), lambda i: (0, 0)),
            pl.BlockSpec((1, K, 4), lambda i: (i, 0, 0)),
            pl.BlockSpec((1, K, 1), lambda i: (i, 0, 0)),
            pl.BlockSpec((1, 4, P), lambda i: (i, 0, 0)),
            pl.BlockSpec((1, P, C), lambda i: (i, 0, 0)),
            pl.BlockSpec((1, 4, P), lambda i: (i, 0, 0)),
            pl.BlockSpec((1, P, C), lambda i: (i, 0, 0)),
        ],
        out_specs=[
            pl.BlockSpec((1, 1, P), lambda i: (i, 0, 0)),
            pl.BlockSpec((1, 1, P), lambda i: (i, 0, 0)),
            pl.BlockSpec((1, 1, 128), lambda i: (i, 0, 0)),
        ],
        out_shape=[
            jax.ShapeDtypeStruct((B, 1, P), jnp.float32),
            jax.ShapeDtypeStruct((B, 1, P), jnp.float32),
            jax.ShapeDtypeStruct((B, 1, 128), jnp.float32),
        ],
    )(priors_t, boxes, labels3, locs1_t, scores1_t, locs2_t, scores2_t)

    o1, o2 = pl.pallas_call(
        _hardneg_body,
        in_specs=[
            pl.BlockSpec((B, P), lambda: (0, 0)),
            pl.BlockSpec((B, P), lambda: (0, 0)),
            pl.BlockSpec((B, 128), lambda: (0, 0)),
        ],
        out_specs=[
            pl.BlockSpec((1, 1), lambda: (0, 0)),
            pl.BlockSpec((1, 1), lambda: (0, 0)),
        ],
        out_shape=[
            jax.ShapeDtypeStruct((1, 1), jnp.float32),
            jax.ShapeDtypeStruct((1, 1), jnp.float32),
        ],
    )(cn1.reshape(B, P), cn2.reshape(B, P), part.reshape(B, 128))

    return (o1.reshape(()), o2.reshape(()))


# speedup vs baseline: 1.9257x; 1.9257x over previous
import jax
import jax.numpy as jnp
from jax import lax
from jax.experimental import pallas as pl
from jax.experimental.pallas import tpu as pltpu

_B, _P, _C, _K = 32, 8732, 21, 12
_THRESHOLD = 0.5
_NEG_POS_RATIO = 3
_ALPHA = 1.0


def _match_body(priors_ref, boxes_ref, labels_ref, label_ref, obj_ref):
    K, P = _K, _P

    pcx = priors_ref[0:1, :]
    pcy = priors_ref[1:2, :]
    pw = priors_ref[2:3, :]
    ph = priors_ref[3:4, :]
    px0 = pcx - pw * 0.5
    py0 = pcy - ph * 0.5
    px1 = pcx + pw * 0.5
    py1 = pcy + ph * 0.5

    b = boxes_ref[0]
    labs = labels_ref[0]
    bx0 = b[:, 0:1]
    by0 = b[:, 1:2]
    bx1 = b[:, 2:3]
    by1 = b[:, 3:4]

    iw = jnp.maximum(jnp.minimum(bx1, px1) - jnp.maximum(bx0, px0), 0.0)
    ih = jnp.maximum(jnp.minimum(by1, py1) - jnp.maximum(by0, py0), 0.0)
    inter = iw * ih
    area_b = (bx1 - bx0) * (by1 - by0)
    area_p = (px1 - px0) * (py1 - py0)
    ov = inter / (area_b + area_p - inter)

    kiota = lax.broadcasted_iota(jnp.int32, (K, P), 0)
    piota = lax.broadcasted_iota(jnp.int32, (K, P), 1)

    ofep = jnp.max(ov, axis=0, keepdims=True)
    obj = jnp.min(jnp.where(ov == ofep, kiota, K), axis=0, keepdims=True)
    mk = jnp.max(ov, axis=1, keepdims=True)
    pfeo = jnp.min(jnp.where(ov == mk, piota, P), axis=1, keepdims=True)
    win = jnp.max(jnp.where(pfeo == piota, kiota, -1), axis=0, keepdims=True)
    obj = jnp.where(win >= 0, win, obj)
    ovf = jnp.where(win >= 0, jnp.full_like(ofep, 1.0), ofep)

    onehot = obj == kiota
    lab_sel = jnp.sum(jnp.where(onehot, labs, 0), axis=0, keepdims=True)
    label = jnp.where(ovf < _THRESHOLD, 0, lab_sel)
    label_ref[0] = label
    obj_ref[0] = obj


def _ce_body(priors_ref, boxes_ref, label_ref, obj_ref,
             locs1_ref, scores1_ref, locs2_ref, scores2_ref,
             cn1_ref, cn2_ref, part_ref):
    f32 = jnp.float32
    K, P, C = _K, _P, _C

    pcx = priors_ref[0:1, :]
    pcy = priors_ref[1:2, :]
    pw = priors_ref[2:3, :]
    ph = priors_ref[3:4, :]

    b = boxes_ref[0]
    bx0 = b[:, 0:1]
    by0 = b[:, 1:2]
    bx1 = b[:, 2:3]
    by1 = b[:, 3:4]

    label = label_ref[0]
    obj = obj_ref[0]
    pos = label > 0
    posf = pos.astype(f32)
    npos = jnp.sum(posf)

    kiota = lax.broadcasted_iota(jnp.int32, (K, P), 0)
    onehotf = (obj == kiota).astype(f32)
    gx0 = jnp.sum(onehotf * bx0, axis=0, keepdims=True)
    gy0 = jnp.sum(onehotf * by0, axis=0, keepdims=True)
    gx1 = jnp.sum(onehotf * bx1, axis=0, keepdims=True)
    gy1 = jnp.sum(onehotf * by1, axis=0, keepdims=True)
    cx = (gx0 + gx1) * 0.5
    cy = (gy0 + gy1) * 0.5
    w = gx1 - gx0
    h = gy1 - gy0
    tl0 = (cx - pcx) / (pw / 10.0)
    tl1 = (cy - pcy) / (ph / 10.0)
    tl2 = jnp.log(w / pw) * 5.0
    tl3 = jnp.log(h / ph) * 5.0

    ciota = lax.broadcasted_iota(jnp.int32, (C, P), 0)

    def branch(locs_ref, scores_ref, cn_ref):
        loc_abs = (jnp.abs(locs_ref[0, 0:1, :] - tl0)
                   + jnp.abs(locs_ref[0, 1:2, :] - tl1)
                   + jnp.abs(locs_ref[0, 2:3, :] - tl2)
                   + jnp.abs(locs_ref[0, 3:4, :] - tl3))
        loc_sum = jnp.sum(loc_abs * posf)
        s = scores_ref[0]
        mx = jnp.max(s, axis=0, keepdims=True)
        lse = jnp.log(jnp.sum(jnp.exp(s - mx), axis=0, keepdims=True)) + mx
        strue = jnp.sum(jnp.where(ciota == label, s, 0.0), axis=0, keepdims=True)
        ce = lse - strue
        cep = jnp.sum(ce * posf)
        cn = jnp.maximum(jnp.where(pos, 0.0, ce), 0.0)
        cn_ref[0] = cn
        return loc_sum, cep

    l1, c1 = branch(locs1_ref, scores1_ref, cn1_ref)
    l2, c2 = branch(locs2_ref, scores2_ref, cn2_ref)

    lane = lax.broadcasted_iota(jnp.int32, (1, 128), 1)
    row = (npos * (lane == 0).astype(f32)
           + l1 * (lane == 1).astype(f32)
           + l2 * (lane == 2).astype(f32)
           + c1 * (lane == 3).astype(f32)
           + c2 * (lane == 4).astype(f32))
    part_ref[0] = row


def _hardneg_body(cn1_ref, cn2_ref, part_ref, out1_ref, out2_ref):
    f32 = jnp.float32
    parts = part_ref[...]
    npos = parts[:, 0:1]
    l1_tot = jnp.sum(parts[:, 1:2])
    l2_tot = jnp.sum(parts[:, 2:3])
    c1_tot = jnp.sum(parts[:, 3:4])
    c2_tot = jnp.sum(parts[:, 4:5])
    np_tot = jnp.sum(npos)
    m = npos * float(_NEG_POS_RATIO)

    def topm_sum(v):
        t = jnp.zeros((_B, 1), jnp.int32)
        for bit in range(30, -1, -1):
            cand = t | (1 << bit)
            tf = lax.bitcast_convert_type(cand, f32)
            cnt = jnp.sum((v >= tf).astype(f32), axis=1, keepdims=True)
            t = jnp.where(cnt >= m, cand, t)
        tf = lax.bitcast_convert_type(t, f32)
        gtf = (v > tf).astype(f32)
        cnt_gt = jnp.sum(gtf, axis=1, keepdims=True)
        hard = jnp.sum(v * gtf, axis=1, keepdims=True) + (m - cnt_gt) * tf
        return jnp.sum(hard)

    h1 = topm_sum(cn1_ref[...])
    h2 = topm_sum(cn2_ref[...])
    o1 = (h1 + c1_tot) / np_tot + _ALPHA * l1_tot / (np_tot * 4.0)
    o2 = (h2 + c2_tot) / np_tot + _ALPHA * l2_tot / (np_tot * 4.0)
    out1_ref[...] = o1.reshape(1, 1)
    out2_ref[...] = o2.reshape(1, 1)


def kernel(predicted_locs1, predicted_scores1, predicted_locs2,
           predicted_scores2, boxes, labels, priors_cxcy):
    B, P, C, K = _B, _P, _C, _K
    priors_t = priors_cxcy.T
    locs1_t = jnp.transpose(predicted_locs1, (0, 2, 1))
    locs2_t = jnp.transpose(predicted_locs2, (0, 2, 1))
    scores1_t = jnp.transpose(predicted_scores1, (0, 2, 1))
    scores2_t = jnp.transpose(predicted_scores2, (0, 2, 1))
    labels3 = labels.astype(jnp.int32).reshape(B, K, 1)

    label_bp, obj_bp = pl.pallas_call(
        _match_body,
        grid=(B,),
        in_specs=[
            pl.BlockSpec((4, P), lambda i: (0, 0)),
            pl.BlockSpec((1, K, 4), lambda i: (i, 0, 0)),
            pl.BlockSpec((1, K, 1), lambda i: (i, 0, 0)),
        ],
        out_specs=[
            pl.BlockSpec((1, 1, P), lambda i: (i, 0, 0)),
            pl.BlockSpec((1, 1, P), lambda i: (i, 0, 0)),
        ],
        out_shape=[
            jax.ShapeDtypeStruct((B, 1, P), jnp.int32),
            jax.ShapeDtypeStruct((B, 1, P), jnp.int32),
        ],
    )(priors_t, boxes, labels3)

    cn1, cn2, part = pl.pallas_call(
        _ce_body,
        grid=(B,),
        in_specs=[
            pl.BlockSpec((4, P), lambda i: (0, 0)),
            pl.BlockSpec((1, K, 4), lambda i: (i, 0, 0)),
            pl.BlockSpec((1, 1, P), lambda i: (i, 0, 0)),
            pl.BlockSpec((1, 1, P), lambda i: (i, 0, 0)),
            pl.BlockSpec((1, 4, P), lambda i: (i, 0, 0)),
            pl.BlockSpec((1, C, P), lambda i: (i, 0, 0)),
            pl.BlockSpec((1, 4, P), lambda i: (i, 0, 0)),
            pl.BlockSpec((1, C, P), lambda i: (i, 0, 0)),
        ],
        out_specs=[
            pl.BlockSpec((1, 1, P), lambda i: (i, 0, 0)),
            pl.BlockSpec((1, 1, P), lambda i: (i, 0, 0)),
            pl.BlockSpec((1, 1, 128), lambda i: (i, 0, 0)),
        ],
        out_shape=[
            jax.ShapeDtypeStruct((B, 1, P), jnp.float32),
            jax.ShapeDtypeStruct((B, 1, P), jnp.float32),
            jax.ShapeDtypeStruct((B, 1, 128), jnp.float32),
        ],
    )(priors_t, boxes, label_bp, obj_bp, locs1_t, scores1_t, locs2_t, scores2_t)

    o1, o2 = pl.pallas_call(
        _hardneg_body,
        in_specs=[
            pl.BlockSpec((B, P), lambda: (0, 0)),
            pl.BlockSpec((B, P), lambda: (0, 0)),
            pl.BlockSpec((B, 128), lambda: (0, 0)),
        ],
        out_specs=[
            pl.BlockSpec((1, 1), lambda: (0, 0)),
            pl.BlockSpec((1, 1), lambda: (0, 0)),
        ],
        out_shape=[
            jax.ShapeDtypeStruct((1, 1), jnp.float32),
            jax.ShapeDtypeStruct((1, 1), jnp.float32),
        ],
    )(cn1.reshape(B, P), cn2.reshape(B, P), part.reshape(B, 128))

    return (o1.reshape(()), o2.reshape(()))
